# baseline (speedup 1.0000x reference)
import jax
import jax.numpy as jnp
from jax import lax
from jax.experimental import pallas as pl
from jax.experimental.pallas import tpu as pltpu
from jax.experimental.pallas import tpu_sc as plsc

BATCH = 16384
FACTORS = 32
LANES = 16
NUM_CORES = 2
NUM_SUBCORES = 16
NW = NUM_CORES * NUM_SUBCORES
B_PER_W = BATCH // NW
CHUNK = 128
N_CHUNKS = B_PER_W // CHUNK
GROUPS = B_PER_W // LANES


def _body(users_r, items_r, user_table, item_table, out_hbm,
          uidx_v, iidx_v, urows_v, irows_v, tbuf_v, out_v, sem):
    wid = lax.axis_index("s") * NUM_CORES + lax.axis_index("c")

    pltpu.sync_copy(users_r.at[wid], uidx_v)
    pltpu.sync_copy(items_r.at[wid], iidx_v)

    copies = []
    for c in range(N_CHUNKS):
        dst_u = urows_v.at[pl.ds(c * CHUNK, CHUNK)]
        dst_i = irows_v.at[pl.ds(c * CHUNK, CHUNK)]
        copies.append(pltpu.async_copy(user_table.at[uidx_v.at[c]], dst_u, sem))
        copies.append(pltpu.async_copy(item_table.at[iidx_v.at[c]], dst_i, sem))
    for cp in copies:
        cp.wait()

    lane_iota = lax.iota(jnp.int32, LANES)

    def group(g, _):
        base = g * LANES
        for j in range(LANES):
            u_row = urows_v[base + j, :]
            i_row = irows_v[base + j, :]
            u0, u1 = plsc.unpack(u_row, format=plsc.PackFormat.INTERLEAVED,
                                 preferred_element_type=jnp.float32)
            v0, v1 = plsc.unpack(i_row, format=plsc.PackFormat.INTERLEAVED,
                                 preferred_element_type=jnp.float32)
            tbuf_v[pl.ds(j * LANES, LANES)] = u0 * v0 + u1 * v1
        acc = jnp.zeros((LANES,), jnp.float32)
        for l in range(LANES):
            acc = acc + plsc.load_gather(tbuf_v, [lane_iota * LANES + l])
        out_v[pl.ds(base, LANES)] = acc
        return 0

    lax.fori_loop(0, GROUPS, group, 0)

    pltpu.sync_copy(out_v, out_hbm.at[pl.ds(wid * B_PER_W, B_PER_W)])


@jax.jit
def kernel(users_id, items_id, user_table, item_table):
    users_r = users_id.reshape(NW, N_CHUNKS, CHUNK)
    items_r = items_id.reshape(NW, N_CHUNKS, CHUNK)
    ut16 = user_table.astype(jnp.bfloat16)
    it16 = item_table.astype(jnp.bfloat16)

    mesh = plsc.VectorSubcoreMesh(
        core_axis_name="c", subcore_axis_name="s",
        num_cores=NUM_CORES, num_subcores=NUM_SUBCORES)

    run = pl.kernel(
        _body,
        out_type=jax.ShapeDtypeStruct((BATCH,), jnp.float32),
        mesh=mesh,
        compiler_params=pltpu.CompilerParams(
            needs_layout_passes=False, use_tc_tiling_on_sc=False),
        scratch_types=[
            pltpu.VMEM((N_CHUNKS, CHUNK), jnp.int32),
            pltpu.VMEM((N_CHUNKS, CHUNK), jnp.int32),
            pltpu.VMEM((B_PER_W, FACTORS), jnp.bfloat16),
            pltpu.VMEM((B_PER_W, FACTORS), jnp.bfloat16),
            pltpu.VMEM((LANES * LANES,), jnp.float32),
            pltpu.VMEM((B_PER_W,), jnp.float32),
            pltpu.SemaphoreType.DMA,
        ],
    )
    return run(users_r, items_r, ut16, it16)


# speedup vs baseline: 5.0302x; 5.0302x over previous
import jax
import jax.numpy as jnp
from jax import lax
from jax.experimental import pallas as pl
from jax.experimental.pallas import tpu as pltpu
from jax.experimental.pallas import tpu_sc as plsc

BATCH = 16384
FACTORS = 32
LANES = 16
NUM_CORES = 2
NUM_SUBCORES = 16
NW = NUM_CORES * NUM_SUBCORES
B_PER_W = BATCH // NW
GROUPS = B_PER_W // LANES
TCOL = 128
NROWS = 1000001
LAST_TC = ((NROWS - TCOL) // TCOL) * TCOL
TAIL_BASE = LAST_TC + TCOL


def _gather_table(tab, idx_v, rows_v, ring_v, tail_v, sem):
    rid = lax.iota(jnp.int32, LANES)

    def group(g, _):
        base = g * LANES
        idx16 = idx_v[pl.ds(base, LANES)]
        vs = [idx16[l] for l in range(LANES)]
        tcs = [pl.multiple_of(jnp.minimum((v >> 7) << 7, LAST_TC), TCOL)
               for v in vs]
        for l in range(LANES):
            pltpu.async_copy(
                tab.at[:, pl.ds(tcs[l], TCOL)], ring_v.at[l], sem)
        for l in range(LANES):
            pltpu.make_async_copy(
                tab.at[:, pl.ds(0, TCOL)], ring_v.at[l], sem).wait()
        for l in range(LANES):
            off = g * (LANES * FACTORS) + l * FACTORS
            lane_in = jnp.minimum(vs[l] - tcs[l], TCOL - 1)
            lane = jnp.full((LANES,), lane_in, jnp.int32)
            slot = jnp.full((LANES,), l, jnp.int32)
            lo = plsc.load_gather(ring_v, [slot, rid, lane])
            hi = plsc.load_gather(ring_v, [slot, rid + LANES, lane])
            rows_v[pl.ds(off, LANES)] = lo
            rows_v[pl.ds(off + LANES, LANES)] = hi

            @pl.when(vs[l] >= TAIL_BASE)
            def _():
                lane_t = jnp.full((LANES,), vs[l] - TAIL_BASE, jnp.int32)
                rows_v[pl.ds(off, LANES)] = (
                    plsc.load_gather(tail_v, [rid, lane_t]))
                rows_v[pl.ds(off + LANES, LANES)] = (
                    plsc.load_gather(tail_v, [rid + LANES, lane_t]))
        return 0

    lax.fori_loop(0, GROUPS, group, 0)


def _body(users_r, items_r, ut, it, utail, itail, out_hbm,
          uidx_v, iidx_v, ring_v, urows_v, irows_v,
          utail_v, itail_v, tbuf_v, out_v, sem):
    wid = lax.axis_index("s") * NUM_CORES + lax.axis_index("c")

    pltpu.sync_copy(users_r.at[wid], uidx_v)
    pltpu.sync_copy(items_r.at[wid], iidx_v)
    pltpu.sync_copy(utail, utail_v)
    pltpu.sync_copy(itail, itail_v)

    _gather_table(ut, uidx_v, urows_v, ring_v, utail_v, sem)
    _gather_table(it, iidx_v, irows_v, ring_v, itail_v, sem)

    lane_iota = lax.iota(jnp.int32, LANES)

    def group(g, _):
        base = g * LANES
        for j in range(LANES):
            off = base * FACTORS + j * FACTORS
            u0 = urows_v[pl.ds(off, LANES)]
            u1 = urows_v[pl.ds(off + LANES, LANES)]
            v0 = irows_v[pl.ds(off, LANES)]
            v1 = irows_v[pl.ds(off + LANES, LANES)]
            tbuf_v[pl.ds(j * LANES, LANES)] = u0 * v0 + u1 * v1
        acc = jnp.zeros((LANES,), jnp.float32)
        for l in range(LANES):
            acc = acc + plsc.load_gather(tbuf_v, [lane_iota * LANES + l])
        out_v[pl.ds(base, LANES)] = acc
        return 0

    lax.fori_loop(0, GROUPS, group, 0)

    pltpu.sync_copy(out_v, out_hbm.at[pl.ds(wid * B_PER_W, B_PER_W)])


@jax.jit
def kernel(users_id, items_id, user_table, item_table):
    users_r = users_id.reshape(NW, B_PER_W)
    items_r = items_id.reshape(NW, B_PER_W)
    npad = TAIL_BASE + TCOL - NROWS
    utail = jnp.pad(user_table[TAIL_BASE:, :], ((0, npad), (0, 0))).T
    itail = jnp.pad(item_table[TAIL_BASE:, :], ((0, npad), (0, 0))).T

    mesh = plsc.VectorSubcoreMesh(
        core_axis_name="c", subcore_axis_name="s",
        num_cores=NUM_CORES, num_subcores=NUM_SUBCORES)

    run = pl.kernel(
        _body,
        out_type=jax.ShapeDtypeStruct((BATCH,), jnp.float32),
        mesh=mesh,
        compiler_params=pltpu.CompilerParams(
            needs_layout_passes=False, use_tc_tiling_on_sc=True),
        scratch_types=[
            pltpu.VMEM((B_PER_W,), jnp.int32),
            pltpu.VMEM((B_PER_W,), jnp.int32),
            pltpu.VMEM((LANES, FACTORS, TCOL), jnp.float32),
            pltpu.VMEM((B_PER_W * FACTORS,), jnp.float32),
            pltpu.VMEM((B_PER_W * FACTORS,), jnp.float32),
            pltpu.VMEM((FACTORS, TCOL), jnp.float32),
            pltpu.VMEM((FACTORS, TCOL), jnp.float32),
            pltpu.VMEM((LANES * LANES,), jnp.float32),
            pltpu.VMEM((B_PER_W,), jnp.float32),
            pltpu.SemaphoreType.DMA,
        ],
    )
    return run(users_r, items_r, user_table.T, item_table.T, utail, itail)


# speedup vs baseline: 5.8037x; 1.1538x over previous
import jax
import jax.numpy as jnp
from jax import lax
from jax.experimental import pallas as pl
from jax.experimental.pallas import tpu as pltpu
from jax.experimental.pallas import tpu_sc as plsc

BATCH = 16384
FACTORS = 32
LANES = 16
NUM_CORES = 2
NUM_SUBCORES = 16
NW = NUM_CORES * NUM_SUBCORES
B_PER_W = BATCH // NW
GROUPS = B_PER_W // LANES
TCOL = 128
NROWS = 1000001
LAST_TC = ((NROWS - TCOL) // TCOL) * TCOL
TAIL_BASE = LAST_TC + TCOL


def _fire(tab, v, ring_v, slot, sem):
    tc = pl.multiple_of(jnp.minimum((v >> 7) << 7, LAST_TC), TCOL)
    pltpu.async_copy(tab.at[:, pl.ds(tc, TCOL)], ring_v.at[slot], sem.at[slot])


def _gather_table(tab, idx_v, rows_v, ring_v, tail_v, sem):
    rid = lax.iota(jnp.int32, LANES)

    idx0 = idx_v[pl.ds(0, LANES)]
    for l in range(LANES):
        _fire(tab, idx0[l], ring_v, l, sem)

    def group(g, _):
        base = g * LANES
        idx16 = idx_v[pl.ds(base, LANES)]
        nxt = jnp.minimum(g + 1, GROUPS - 1) * LANES
        idxn = idx_v[pl.ds(nxt, LANES)]
        for l in range(LANES):
            v = idx16[l]
            pltpu.make_async_copy(
                tab.at[:, pl.ds(0, TCOL)], ring_v.at[l], sem.at[l]).wait()
            tc = jnp.minimum((v >> 7) << 7, LAST_TC)
            off = g * (LANES * FACTORS) + l * FACTORS
            lane = jnp.full((LANES,), jnp.minimum(v - tc, TCOL - 1), jnp.int32)
            slot = jnp.full((LANES,), l, jnp.int32)
            lo = plsc.load_gather(ring_v, [slot, rid, lane])
            hi = plsc.load_gather(ring_v, [slot, rid + LANES, lane])
            rows_v[pl.ds(off, LANES)] = lo
            rows_v[pl.ds(off + LANES, LANES)] = hi

            @pl.when(v >= TAIL_BASE)
            def _():
                lane_t = jnp.full((LANES,), v - TAIL_BASE, jnp.int32)
                rows_v[pl.ds(off, LANES)] = (
                    plsc.load_gather(tail_v, [rid, lane_t]))
                rows_v[pl.ds(off + LANES, LANES)] = (
                    plsc.load_gather(tail_v, [rid + LANES, lane_t]))

            @pl.when(g < GROUPS - 1)
            def _():
                _fire(tab, idxn[l], ring_v, l, sem)
        return 0

    lax.fori_loop(0, GROUPS, group, 0)


def _body(users_r, items_r, ut, it, utail, itail, out_hbm,
          uidx_v, iidx_v, ring_v, urows_v, irows_v,
          utail_v, itail_v, tbuf_v, out_v, sem):
    wid = lax.axis_index("s") * NUM_CORES + lax.axis_index("c")

    pltpu.sync_copy(users_r.at[wid], uidx_v)
    pltpu.sync_copy(items_r.at[wid], iidx_v)
    pltpu.sync_copy(utail, utail_v)
    pltpu.sync_copy(itail, itail_v)

    _gather_table(ut, uidx_v, urows_v, ring_v, utail_v, sem)
    _gather_table(it, iidx_v, irows_v, ring_v, itail_v, sem)

    lane_iota = lax.iota(jnp.int32, LANES)

    def group(g, _):
        base = g * LANES
        for j in range(LANES):
            off = base * FACTORS + j * FACTORS
            u0 = urows_v[pl.ds(off, LANES)]
            u1 = urows_v[pl.ds(off + LANES, LANES)]
            v0 = irows_v[pl.ds(off, LANES)]
            v1 = irows_v[pl.ds(off + LANES, LANES)]
            tbuf_v[pl.ds(j * LANES, LANES)] = u0 * v0 + u1 * v1
        acc = jnp.zeros((LANES,), jnp.float32)
        for l in range(LANES):
            acc = acc + plsc.load_gather(tbuf_v, [lane_iota * LANES + l])
        out_v[pl.ds(base, LANES)] = acc
        return 0

    lax.fori_loop(0, GROUPS, group, 0)

    pltpu.sync_copy(out_v, out_hbm.at[pl.ds(wid * B_PER_W, B_PER_W)])


@jax.jit
def kernel(users_id, items_id, user_table, item_table):
    users_r = users_id.reshape(NW, B_PER_W)
    items_r = items_id.reshape(NW, B_PER_W)
    npad = TAIL_BASE + TCOL - NROWS
    utail = jnp.pad(user_table[TAIL_BASE:, :], ((0, npad), (0, 0))).T
    itail = jnp.pad(item_table[TAIL_BASE:, :], ((0, npad), (0, 0))).T

    mesh = plsc.VectorSubcoreMesh(
        core_axis_name="c", subcore_axis_name="s",
        num_cores=NUM_CORES, num_subcores=NUM_SUBCORES)

    run = pl.kernel(
        _body,
        out_type=jax.ShapeDtypeStruct((BATCH,), jnp.float32),
        mesh=mesh,
        compiler_params=pltpu.CompilerParams(
            needs_layout_passes=False, use_tc_tiling_on_sc=True),
        scratch_types=[
            pltpu.VMEM((B_PER_W,), jnp.int32),
            pltpu.VMEM((B_PER_W,), jnp.int32),
            pltpu.VMEM((LANES, FACTORS, TCOL), jnp.float32),
            pltpu.VMEM((B_PER_W * FACTORS,), jnp.float32),
            pltpu.VMEM((B_PER_W * FACTORS,), jnp.float32),
            pltpu.VMEM((FACTORS, TCOL), jnp.float32),
            pltpu.VMEM((FACTORS, TCOL), jnp.float32),
            pltpu.VMEM((LANES * LANES,), jnp.float32),
            pltpu.VMEM((B_PER_W,), jnp.float32),
            pltpu.SemaphoreType.DMA((LANES,)),
        ],
    )
    return run(users_r, items_r, user_table.T, item_table.T, utail, itail)
